# baseline (device time: 80667 ns/iter reference)
import jax
import jax.numpy as jnp
from jax import lax
from jax.experimental import pallas as pl
from jax.experimental.pallas import tpu as pltpu

W = 32
G = 4
S = 8
Q = 6
T = 3

PERM = (0, 1, 2, 5, 6, 7, 4, 3)
INV = (0, 1, 2, 7, 6, 3, 4, 5)


def _gelu(z):
    return 0.5 * z * (1.0 + jnp.tanh(0.7978845608 * (z + 0.044715 * z * z * z)))


def kernel(A, B):
    m, k = A.shape
    _, n = B.shape
    slab = m // S
    half = slab // 2
    qr = half // Q
    sub = slab // G
    th = sub // T

    perm_packed = 0
    inv_packed = 0
    for i in range(S):
        perm_packed |= PERM[i] << (4 * i)
        inv_packed |= INV[i] << (4 * i)

    def body(a_ref, b_ref, out_ref, p_ref, recv1r_ref, recv1l_ref,
             recv2_ref,
             rs1r_send, rs1r_recv, rs1l_send, rs1l_recv,
             rs2_send, rs2_recv, ag2_send, ag2_recv,
             ag1r_send, ag1r_recv, ag1l_send, ag1l_recv):
        r = lax.axis_index("i")
        gg = r // S
        jj = lax.rem(r, S)

        def nib(packed, idx):
            return lax.bitwise_and(
                lax.shift_right_logical(jnp.int32(packed), 4 * idx),
                jnp.int32(0xF),
            )

        q = nib(inv_packed, jj)
        right1 = gg * S + nib(perm_packed, lax.rem(q + 1, S))
        left1 = gg * S + nib(perm_packed, lax.rem(q + S - 1, S))
        right2 = lax.rem(gg + 1, G) * S + jj
        left2 = lax.rem(gg + G - 1, G) * S + jj

        barrier_sem = pltpu.get_barrier_semaphore()
        for nbr in (left1, right1, left2, right2):
            pl.semaphore_signal(
                barrier_sem, inc=1,
                device_id=(nbr,), device_id_type=pl.DeviceIdType.MESH,
            )
        pl.semaphore_wait(barrier_sem, 4)

        bmat = b_ref[...].astype(jnp.bfloat16)

        def compute_slab(idx):
            rows = pl.ds(lax.rem(idx + 2 * S, S) * slab, slab)
            p_ref[rows, :] = jnp.dot(
                a_ref[rows, :].astype(jnp.bfloat16), bmat,
                preferred_element_type=jnp.float32,
            ).astype(jnp.bfloat16)

        pending = []

        def remote_copy(src, dst, send_sem, recv_sem, dev):
            d = pltpu.make_async_remote_copy(
                src_ref=src, dst_ref=dst, send_sem=send_sem,
                recv_sem=recv_sem, device_id=(dev,),
                device_id_type=pl.DeviceIdType.MESH,
            )
            d.start()
            pending.append(d)
            return d

        slab_schedule = {
            1: (q - 2, q + 2),
            2: (q - 3, q + 3),
            3: (q + 4,),
            4: (q,),
        }

        compute_slab(q - 1)
        compute_slab(q + 1)
        hops_r = []
        hops_l = []
        for h in range(S - 1):
            for idx in slab_schedule.get(h, ()):
                compute_slab(idx)
            sr = lax.rem(q + S - h - 1, S)
            sl = lax.rem(q + h + 1, S)
            dr, dl = [], []
            for u in range(Q):
                top = pl.ds(sr * slab + u * qr, qr)
                bot = pl.ds(sl * slab + half + u * qr, qr)
                if h > 0:
                    hops_r[h - 1][u].wait_recv()
                    recv1r_ref[h - 1, u] = recv1r_ref[h - 1, u] + p_ref[top, :]
                    src_r = recv1r_ref.at[h - 1, u]
                else:
                    src_r = p_ref.at[top, :]
                dr.append(remote_copy(
                    src_r, recv1r_ref.at[h, u],
                    rs1r_send.at[h, u], rs1r_recv.at[h, u], right1))
                if h > 0:
                    hops_l[h - 1][u].wait_recv()
                    recv1l_ref[h - 1, u] = recv1l_ref[h - 1, u] + p_ref[bot, :]
                    src_l = recv1l_ref.at[h - 1, u]
                else:
                    src_l = p_ref.at[bot, :]
                dl.append(remote_copy(
                    src_l, recv1l_ref.at[h, u],
                    rs1l_send.at[h, u], rs1l_recv.at[h, u], left1))
            hops_r.append(dr)
            hops_l.append(dl)
        for u in range(Q):
            hops_r[S - 2][u].wait_recv()
            hops_l[S - 2][u].wait_recv()
            top = pl.ds(q * slab + u * qr, qr)
            bot = pl.ds(q * slab + half + u * qr, qr)
            p_ref[top, :] = p_ref[top, :] + recv1r_ref[S - 2, u]
            p_ref[bot, :] = p_ref[bot, :] + recv1l_ref[S - 2, u]

        base = q * slab
        rs2 = []
        for h in range(G - 1):
            s2 = lax.rem(gg + G - h - 1, G)
            dt = []
            for t in range(T):
                rows = pl.ds(base + s2 * sub + t * th, th)
                if h > 0:
                    rs2[h - 1][t].wait_recv()
                    recv2_ref[h - 1, t] = recv2_ref[h - 1, t] + p_ref[rows, :]
                    src = recv2_ref.at[h - 1, t]
                else:
                    src = p_ref.at[rows, :]
                dt.append(remote_copy(
                    src, recv2_ref.at[h, t],
                    rs2_send.at[h, t], rs2_recv.at[h, t], right2))
            rs2.append(dt)

        ag2 = [[], [], []]
        for t in range(T):
            rs2[G - 2][t].wait_recv()
            rows = pl.ds(base + gg * sub + t * th, th)
            z = (
                p_ref[rows, :].astype(jnp.float32)
                + recv2_ref[G - 2, t].astype(jnp.float32)
            )
            out_ref[rows, :] = _gelu(z).astype(jnp.bfloat16)
            ag2[0].append(remote_copy(
                out_ref.at[rows, :], out_ref.at[rows, :],
                ag2_send.at[0, t], ag2_recv.at[0, t], right2))

        for hh in range(1, G - 1):
            c2 = lax.rem(gg + G - hh, G)
            for t in range(T):
                rows = pl.ds(base + c2 * sub + t * th, th)
                ag2[hh - 1][t].wait_recv()
                ag2[hh].append(remote_copy(
                    out_ref.at[rows, :], out_ref.at[rows, :],
                    ag2_send.at[hh, t], ag2_recv.at[hh, t], right2))
        for t in range(T):
            ag2[G - 2][t].wait_recv()

        agr = []
        agl = []
        for hh in range(S - 1):
            cr = lax.rem(q + S - hh, S)
            cl = lax.rem(q + hh, S)
            dr, dl = [], []
            for u in range(Q):
                top = pl.ds(cr * slab + u * qr, qr)
                bot = pl.ds(cl * slab + half + u * qr, qr)
                if hh > 0:
                    agr[hh - 1][u].wait_recv()
                    agl[hh - 1][u].wait_recv()
                dr.append(remote_copy(
                    out_ref.at[top, :], out_ref.at[top, :],
                    ag1r_send.at[hh, u], ag1r_recv.at[hh, u], right1))
                dl.append(remote_copy(
                    out_ref.at[bot, :], out_ref.at[bot, :],
                    ag1l_send.at[hh, u], ag1l_recv.at[hh, u], left1))
            agr.append(dr)
            agl.append(dl)
        for u in range(Q):
            agr[S - 2][u].wait_recv()
            agl[S - 2][u].wait_recv()

        for d in pending:
            d.wait_send()

    return pl.pallas_call(
        body,
        out_shape=jax.ShapeDtypeStruct((m, n), jnp.bfloat16),
        in_specs=[
            pl.BlockSpec(memory_space=pltpu.VMEM),
            pl.BlockSpec(memory_space=pltpu.VMEM),
        ],
        out_specs=pl.BlockSpec(memory_space=pltpu.VMEM),
        scratch_shapes=[
            pltpu.VMEM((m, n), jnp.bfloat16),
            pltpu.VMEM((S - 1, Q, qr, n), jnp.bfloat16),
            pltpu.VMEM((S - 1, Q, qr, n), jnp.bfloat16),
            pltpu.VMEM((G - 1, T, th, n), jnp.bfloat16),
            pltpu.SemaphoreType.DMA((S - 1, Q)),
            pltpu.SemaphoreType.DMA((S - 1, Q)),
            pltpu.SemaphoreType.DMA((S - 1, Q)),
            pltpu.SemaphoreType.DMA((S - 1, Q)),
            pltpu.SemaphoreType.DMA((G - 1, T)),
            pltpu.SemaphoreType.DMA((G - 1, T)),
            pltpu.SemaphoreType.DMA((G - 1, T)),
            pltpu.SemaphoreType.DMA((G - 1, T)),
            pltpu.SemaphoreType.DMA((S - 1, Q)),
            pltpu.SemaphoreType.DMA((S - 1, Q)),
            pltpu.SemaphoreType.DMA((S - 1, Q)),
            pltpu.SemaphoreType.DMA((S - 1, Q)),
        ],
        compiler_params=pltpu.CompilerParams(collective_id=0),
    )(A, B)


# device time: 79092 ns/iter; 1.0199x vs baseline; 1.0199x over previous
import jax
import jax.numpy as jnp
from jax import lax
from jax.experimental import pallas as pl
from jax.experimental.pallas import tpu as pltpu

W = 32
G = 4
S = 8
Q = 3
T = 3

PERM = (0, 1, 2, 5, 6, 7, 4, 3)
INV = (0, 1, 2, 7, 6, 3, 4, 5)


def _gelu(z):
    return 0.5 * z * (1.0 + jnp.tanh(0.7978845608 * (z + 0.044715 * z * z * z)))


def kernel(A, B):
    m, k = A.shape
    _, n = B.shape
    slab = m // S
    half = slab // 2
    qr = half // Q
    sub = slab // G
    th = sub // T

    perm_packed = 0
    inv_packed = 0
    for i in range(S):
        perm_packed |= PERM[i] << (4 * i)
        inv_packed |= INV[i] << (4 * i)

    def body(a_ref, b_ref, out_ref, p_ref, recv1r_ref, recv1l_ref,
             recv2_ref,
             rs1r_send, rs1r_recv, rs1l_send, rs1l_recv,
             rs2_send, rs2_recv, ag2_send, ag2_recv,
             ag1r_send, ag1r_recv, ag1l_send, ag1l_recv):
        r = lax.axis_index("i")
        gg = r // S
        jj = lax.rem(r, S)

        def nib(packed, idx):
            return lax.bitwise_and(
                lax.shift_right_logical(jnp.int32(packed), 4 * idx),
                jnp.int32(0xF),
            )

        q = nib(inv_packed, jj)
        right1 = gg * S + nib(perm_packed, lax.rem(q + 1, S))
        left1 = gg * S + nib(perm_packed, lax.rem(q + S - 1, S))
        right2 = lax.rem(gg + 1, G) * S + jj
        left2 = lax.rem(gg + G - 1, G) * S + jj

        barrier_sem = pltpu.get_barrier_semaphore()
        for nbr in (left1, right1, left2, right2):
            pl.semaphore_signal(
                barrier_sem, inc=1,
                device_id=(nbr,), device_id_type=pl.DeviceIdType.MESH,
            )
        pl.semaphore_wait(barrier_sem, 4)

        bmat = b_ref[...].astype(jnp.bfloat16)

        def compute_slab(idx):
            rows = pl.ds(lax.rem(idx + 2 * S, S) * slab, slab)
            p_ref[rows, :] = jnp.dot(
                a_ref[rows, :].astype(jnp.bfloat16), bmat,
                preferred_element_type=jnp.float32,
            ).astype(jnp.bfloat16)

        pending = []

        def remote_copy(src, dst, send_sem, recv_sem, dev):
            d = pltpu.make_async_remote_copy(
                src_ref=src, dst_ref=dst, send_sem=send_sem,
                recv_sem=recv_sem, device_id=(dev,),
                device_id_type=pl.DeviceIdType.MESH,
            )
            d.start()
            pending.append(d)
            return d

        slab_schedule = {
            1: (q - 2, q + 2),
            2: (q - 3, q + 3),
            3: (q + 4,),
            4: (q,),
        }

        compute_slab(q - 1)
        compute_slab(q + 1)
        hops_r = []
        hops_l = []
        for h in range(S - 1):
            for idx in slab_schedule.get(h, ()):
                compute_slab(idx)
            sr = lax.rem(q + S - h - 1, S)
            sl = lax.rem(q + h + 1, S)
            dr, dl = [], []
            for u in range(Q):
                top = pl.ds(sr * slab + u * qr, qr)
                bot = pl.ds(sl * slab + half + u * qr, qr)
                if h > 0:
                    hops_r[h - 1][u].wait_recv()
                    recv1r_ref[h - 1, u] = recv1r_ref[h - 1, u] + p_ref[top, :]
                    src_r = recv1r_ref.at[h - 1, u]
                else:
                    src_r = p_ref.at[top, :]
                dr.append(remote_copy(
                    src_r, recv1r_ref.at[h, u],
                    rs1r_send.at[h, u], rs1r_recv.at[h, u], right1))
                if h > 0:
                    hops_l[h - 1][u].wait_recv()
                    recv1l_ref[h - 1, u] = recv1l_ref[h - 1, u] + p_ref[bot, :]
                    src_l = recv1l_ref.at[h - 1, u]
                else:
                    src_l = p_ref.at[bot, :]
                dl.append(remote_copy(
                    src_l, recv1l_ref.at[h, u],
                    rs1l_send.at[h, u], rs1l_recv.at[h, u], left1))
            hops_r.append(dr)
            hops_l.append(dl)
        for u in range(Q):
            hops_r[S - 2][u].wait_recv()
            hops_l[S - 2][u].wait_recv()
            top = pl.ds(q * slab + u * qr, qr)
            bot = pl.ds(q * slab + half + u * qr, qr)
            p_ref[top, :] = p_ref[top, :] + recv1r_ref[S - 2, u]
            p_ref[bot, :] = p_ref[bot, :] + recv1l_ref[S - 2, u]

        base = q * slab
        rs2 = []
        for h in range(G - 1):
            s2 = lax.rem(gg + G - h - 1, G)
            dt = []
            for t in range(T):
                rows = pl.ds(base + s2 * sub + t * th, th)
                if h > 0:
                    rs2[h - 1][t].wait_recv()
                    recv2_ref[h - 1, t] = recv2_ref[h - 1, t] + p_ref[rows, :]
                    src = recv2_ref.at[h - 1, t]
                else:
                    src = p_ref.at[rows, :]
                dt.append(remote_copy(
                    src, recv2_ref.at[h, t],
                    rs2_send.at[h, t], rs2_recv.at[h, t], right2))
            rs2.append(dt)

        ag2 = [[], [], []]
        for t in range(T):
            rs2[G - 2][t].wait_recv()
            rows = pl.ds(base + gg * sub + t * th, th)
            z = (
                p_ref[rows, :].astype(jnp.float32)
                + recv2_ref[G - 2, t].astype(jnp.float32)
            )
            out_ref[rows, :] = _gelu(z).astype(jnp.bfloat16)
            ag2[0].append(remote_copy(
                out_ref.at[rows, :], out_ref.at[rows, :],
                ag2_send.at[0, t], ag2_recv.at[0, t], right2))

        for hh in range(1, G - 1):
            c2 = lax.rem(gg + G - hh, G)
            for t in range(T):
                rows = pl.ds(base + c2 * sub + t * th, th)
                ag2[hh - 1][t].wait_recv()
                ag2[hh].append(remote_copy(
                    out_ref.at[rows, :], out_ref.at[rows, :],
                    ag2_send.at[hh, t], ag2_recv.at[hh, t], right2))
        for t in range(T):
            ag2[G - 2][t].wait_recv()

        agr = []
        agl = []
        for hh in range(S - 1):
            cr = lax.rem(q + S - hh, S)
            cl = lax.rem(q + hh, S)
            dr, dl = [], []
            for u in range(Q):
                top = pl.ds(cr * slab + u * qr, qr)
                bot = pl.ds(cl * slab + half + u * qr, qr)
                if hh > 0:
                    agr[hh - 1][u].wait_recv()
                    agl[hh - 1][u].wait_recv()
                dr.append(remote_copy(
                    out_ref.at[top, :], out_ref.at[top, :],
                    ag1r_send.at[hh, u], ag1r_recv.at[hh, u], right1))
                dl.append(remote_copy(
                    out_ref.at[bot, :], out_ref.at[bot, :],
                    ag1l_send.at[hh, u], ag1l_recv.at[hh, u], left1))
            agr.append(dr)
            agl.append(dl)
        for u in range(Q):
            agr[S - 2][u].wait_recv()
            agl[S - 2][u].wait_recv()

        for d in pending:
            d.wait_send()

    return pl.pallas_call(
        body,
        out_shape=jax.ShapeDtypeStruct((m, n), jnp.bfloat16),
        in_specs=[
            pl.BlockSpec(memory_space=pltpu.VMEM),
            pl.BlockSpec(memory_space=pltpu.VMEM),
        ],
        out_specs=pl.BlockSpec(memory_space=pltpu.VMEM),
        scratch_shapes=[
            pltpu.VMEM((m, n), jnp.bfloat16),
            pltpu.VMEM((S - 1, Q, qr, n), jnp.bfloat16),
            pltpu.VMEM((S - 1, Q, qr, n), jnp.bfloat16),
            pltpu.VMEM((G - 1, T, th, n), jnp.bfloat16),
            pltpu.SemaphoreType.DMA((S - 1, Q)),
            pltpu.SemaphoreType.DMA((S - 1, Q)),
            pltpu.SemaphoreType.DMA((S - 1, Q)),
            pltpu.SemaphoreType.DMA((S - 1, Q)),
            pltpu.SemaphoreType.DMA((G - 1, T)),
            pltpu.SemaphoreType.DMA((G - 1, T)),
            pltpu.SemaphoreType.DMA((G - 1, T)),
            pltpu.SemaphoreType.DMA((G - 1, T)),
            pltpu.SemaphoreType.DMA((S - 1, Q)),
            pltpu.SemaphoreType.DMA((S - 1, Q)),
            pltpu.SemaphoreType.DMA((S - 1, Q)),
            pltpu.SemaphoreType.DMA((S - 1, Q)),
        ],
        compiler_params=pltpu.CompilerParams(collective_id=0),
    )(A, B)


# device time: 77869 ns/iter; 1.0359x vs baseline; 1.0157x over previous
import jax
import jax.numpy as jnp
from jax import lax
from jax.experimental import pallas as pl
from jax.experimental.pallas import tpu as pltpu

W = 32
G = 4
S = 8
Q = 3
T = 3

PERM = (0, 1, 2, 5, 6, 7, 4, 3)
INV = (0, 1, 2, 7, 6, 3, 4, 5)


def _gelu(z):
    return 0.5 * z * (1.0 + jnp.tanh(0.7978845608 * (z + 0.044715 * z * z * z)))


def kernel(A, B):
    m, k = A.shape
    _, n = B.shape
    slab = m // S
    half = slab // 2
    qr = half // Q
    sub = slab // G
    th = sub // T

    perm_packed = 0
    inv_packed = 0
    for i in range(S):
        perm_packed |= PERM[i] << (4 * i)
        inv_packed |= INV[i] << (4 * i)

    def body(a_ref, b_ref, out_ref, p_ref, recv1r_ref, recv1l_ref,
             recv2_ref,
             rs1r_send, rs1r_recv, rs1l_send, rs1l_recv,
             rs2_send, rs2_recv, ag2_send, ag2_recv,
             ag1r_send, ag1r_recv, ag1l_send, ag1l_recv):
        r = lax.axis_index("i")
        gg = r // S
        jj = lax.rem(r, S)

        def nib(packed, idx):
            return lax.bitwise_and(
                lax.shift_right_logical(jnp.int32(packed), 4 * idx),
                jnp.int32(0xF),
            )

        q = nib(inv_packed, jj)
        right1 = gg * S + nib(perm_packed, lax.rem(q + 1, S))
        left1 = gg * S + nib(perm_packed, lax.rem(q + S - 1, S))
        right2 = lax.rem(gg + 1, G) * S + jj
        left2 = lax.rem(gg + G - 1, G) * S + jj

        barrier_sem = pltpu.get_barrier_semaphore()
        for nbr in (left1, right1, left2, right2):
            pl.semaphore_signal(
                barrier_sem, inc=1,
                device_id=(nbr,), device_id_type=pl.DeviceIdType.MESH,
            )

        bmat = b_ref[...].astype(jnp.bfloat16)

        def compute_slab(idx):
            rows = pl.ds(lax.rem(idx + 2 * S, S) * slab, slab)
            p_ref[rows, :] = jnp.dot(
                a_ref[rows, :].astype(jnp.bfloat16), bmat,
                preferred_element_type=jnp.float32,
            ).astype(jnp.bfloat16)

        pending = []

        def remote_copy(src, dst, send_sem, recv_sem, dev):
            d = pltpu.make_async_remote_copy(
                src_ref=src, dst_ref=dst, send_sem=send_sem,
                recv_sem=recv_sem, device_id=(dev,),
                device_id_type=pl.DeviceIdType.MESH,
            )
            d.start()
            pending.append(d)
            return d

        slab_schedule = {
            1: (q - 2, q + 2),
            2: (q - 3, q + 3),
            3: (q + 4,),
            4: (q,),
        }

        compute_slab(q - 1)
        compute_slab(q + 1)
        pl.semaphore_wait(barrier_sem, 4)
        hops_r = []
        hops_l = []
        for h in range(S - 1):
            for idx in slab_schedule.get(h, ()):
                compute_slab(idx)
            sr = lax.rem(q + S - h - 1, S)
            sl = lax.rem(q + h + 1, S)
            dr, dl = [], []
            for u in range(Q):
                top = pl.ds(sr * slab + u * qr, qr)
                bot = pl.ds(sl * slab + half + u * qr, qr)
                if h > 0:
                    hops_r[h - 1][u].wait_recv()
                    recv1r_ref[h - 1, u] = recv1r_ref[h - 1, u] + p_ref[top, :]
                    src_r = recv1r_ref.at[h - 1, u]
                else:
                    src_r = p_ref.at[top, :]
                dr.append(remote_copy(
                    src_r, recv1r_ref.at[h, u],
                    rs1r_send.at[h, u], rs1r_recv.at[h, u], right1))
                if h > 0:
                    hops_l[h - 1][u].wait_recv()
                    recv1l_ref[h - 1, u] = recv1l_ref[h - 1, u] + p_ref[bot, :]
                    src_l = recv1l_ref.at[h - 1, u]
                else:
                    src_l = p_ref.at[bot, :]
                dl.append(remote_copy(
                    src_l, recv1l_ref.at[h, u],
                    rs1l_send.at[h, u], rs1l_recv.at[h, u], left1))
            hops_r.append(dr)
            hops_l.append(dl)
        for u in range(Q):
            hops_r[S - 2][u].wait_recv()
            hops_l[S - 2][u].wait_recv()
            top = pl.ds(q * slab + u * qr, qr)
            bot = pl.ds(q * slab + half + u * qr, qr)
            p_ref[top, :] = p_ref[top, :] + recv1r_ref[S - 2, u]
            p_ref[bot, :] = p_ref[bot, :] + recv1l_ref[S - 2, u]

        base = q * slab
        rs2 = []
        for h in range(G - 1):
            s2 = lax.rem(gg + G - h - 1, G)
            dt = []
            for t in range(T):
                rows = pl.ds(base + s2 * sub + t * th, th)
                if h > 0:
                    rs2[h - 1][t].wait_recv()
                    recv2_ref[h - 1, t] = recv2_ref[h - 1, t] + p_ref[rows, :]
                    src = recv2_ref.at[h - 1, t]
                else:
                    src = p_ref.at[rows, :]
                dt.append(remote_copy(
                    src, recv2_ref.at[h, t],
                    rs2_send.at[h, t], rs2_recv.at[h, t], right2))
            rs2.append(dt)

        ag2 = [[], [], []]
        for t in range(T):
            rs2[G - 2][t].wait_recv()
            rows = pl.ds(base + gg * sub + t * th, th)
            z = (
                p_ref[rows, :].astype(jnp.float32)
                + recv2_ref[G - 2, t].astype(jnp.float32)
            )
            out_ref[rows, :] = _gelu(z).astype(jnp.bfloat16)
            ag2[0].append(remote_copy(
                out_ref.at[rows, :], out_ref.at[rows, :],
                ag2_send.at[0, t], ag2_recv.at[0, t], right2))

        for hh in range(1, G - 1):
            c2 = lax.rem(gg + G - hh, G)
            for t in range(T):
                rows = pl.ds(base + c2 * sub + t * th, th)
                ag2[hh - 1][t].wait_recv()
                ag2[hh].append(remote_copy(
                    out_ref.at[rows, :], out_ref.at[rows, :],
                    ag2_send.at[hh, t], ag2_recv.at[hh, t], right2))
        for t in range(T):
            ag2[G - 2][t].wait_recv()

        agr = []
        agl = []
        for hh in range(S - 1):
            cr = lax.rem(q + S - hh, S)
            cl = lax.rem(q + hh, S)
            dr, dl = [], []
            for u in range(Q):
                top = pl.ds(cr * slab + u * qr, qr)
                bot = pl.ds(cl * slab + half + u * qr, qr)
                if hh > 0:
                    agr[hh - 1][u].wait_recv()
                    agl[hh - 1][u].wait_recv()
                dr.append(remote_copy(
                    out_ref.at[top, :], out_ref.at[top, :],
                    ag1r_send.at[hh, u], ag1r_recv.at[hh, u], right1))
                dl.append(remote_copy(
                    out_ref.at[bot, :], out_ref.at[bot, :],
                    ag1l_send.at[hh, u], ag1l_recv.at[hh, u], left1))
            agr.append(dr)
            agl.append(dl)
        for u in range(Q):
            agr[S - 2][u].wait_recv()
            agl[S - 2][u].wait_recv()

        for d in pending:
            d.wait_send()

    return pl.pallas_call(
        body,
        out_shape=jax.ShapeDtypeStruct((m, n), jnp.bfloat16),
        in_specs=[
            pl.BlockSpec(memory_space=pltpu.VMEM),
            pl.BlockSpec(memory_space=pltpu.VMEM),
        ],
        out_specs=pl.BlockSpec(memory_space=pltpu.VMEM),
        scratch_shapes=[
            pltpu.VMEM((m, n), jnp.bfloat16),
            pltpu.VMEM((S - 1, Q, qr, n), jnp.bfloat16),
            pltpu.VMEM((S - 1, Q, qr, n), jnp.bfloat16),
            pltpu.VMEM((G - 1, T, th, n), jnp.bfloat16),
            pltpu.SemaphoreType.DMA((S - 1, Q)),
            pltpu.SemaphoreType.DMA((S - 1, Q)),
            pltpu.SemaphoreType.DMA((S - 1, Q)),
            pltpu.SemaphoreType.DMA((S - 1, Q)),
            pltpu.SemaphoreType.DMA((G - 1, T)),
            pltpu.SemaphoreType.DMA((G - 1, T)),
            pltpu.SemaphoreType.DMA((G - 1, T)),
            pltpu.SemaphoreType.DMA((G - 1, T)),
            pltpu.SemaphoreType.DMA((S - 1, Q)),
            pltpu.SemaphoreType.DMA((S - 1, Q)),
            pltpu.SemaphoreType.DMA((S - 1, Q)),
            pltpu.SemaphoreType.DMA((S - 1, Q)),
        ],
        compiler_params=pltpu.CompilerParams(collective_id=0),
    )(A, B)
